# Initial kernel scaffold; baseline (speedup 1.0000x reference)
import jax
import jax.numpy as jnp
from jax.experimental import pallas as pl


def kernel(batch_size, family_ids, mu, log_sigma, noise):
    raise NotImplementedError("write your pallas kernel here")


# speedup vs baseline: 1.2826x; 1.2826x over previous
import functools

import jax
import jax.numpy as jnp
from jax import lax
from jax.experimental import pallas as pl
from jax.experimental.pallas import tpu as pltpu
from jax.experimental.pallas import tpu_sc as plsc

_LN10 = 2.302585092994046

_ROWS = 128
_COLS = 128
_NW = 32
_RPW = _ROWS // _NW
_LANES = 16


def _build():
    mesh = plsc.VectorSubcoreMesh(core_axis_name="c", subcore_axis_name="s")

    @functools.partial(
        pl.kernel,
        mesh=mesh,
        out_type=jax.ShapeDtypeStruct((_ROWS, _COLS), jnp.float32),
        scratch_types=[
            pltpu.VMEM((_RPW, _COLS), jnp.int32),
            pltpu.VMEM((_RPW, _COLS), jnp.float32),
            pltpu.VMEM((_RPW, _COLS), jnp.float32),
            pltpu.VMEM((_RPW, _COLS), jnp.float32),
            pltpu.VMEM((_RPW, _COLS), jnp.float32),
            pltpu.SemaphoreType.DMA,
        ],
    )
    def k(ids_hbm, mu_hbm, ls_hbm, nz_hbm, out_hbm,
          idx_v, mu_v, ls_v, nz_v, out_v, sem):
        wid = lax.axis_index("s") * 2 + lax.axis_index("c")
        base = wid * _RPW
        pltpu.sync_copy(ids_hbm.at[pl.ds(base, _RPW)], idx_v)
        copies = []
        for r in range(_RPW):
            copies.append(pltpu.async_copy(mu_hbm.at[idx_v.at[r]], mu_v.at[r], sem))
            copies.append(pltpu.async_copy(ls_hbm.at[idx_v.at[r]], ls_v.at[r], sem))
        pltpu.sync_copy(nz_hbm.at[pl.ds(base, _RPW)], nz_v)
        for c in copies:
            c.wait()
        for r in range(_RPW):
            for i in range(_COLS // _LANES):
                sl = pl.ds(i * _LANES, _LANES)
                m = mu_v[r, sl]
                s = ls_v[r, sl]
                z = nz_v[r, sl]
                out_v[r, sl] = jnp.exp((m + jnp.exp(s) * z) * _LN10)
        pltpu.sync_copy(out_v, out_hbm.at[pl.ds(base, _RPW)])

    return k


_sc_kernel = _build()


def kernel(batch_size, family_ids, mu, log_sigma, noise):
    ids2 = family_ids.astype(jnp.int32).reshape(_ROWS, _COLS)
    nz2 = noise.reshape(_ROWS, _COLS)
    out = _sc_kernel(ids2, mu, log_sigma, nz2)
    return out.reshape(-1)
